# baseline (speedup 1.0000x reference)
import dataclasses
import functools

import jax
import jax.numpy as jnp
from jax import lax
from jax.experimental import pallas as pl
from jax.experimental.pallas import tpu as pltpu
from jax.experimental.pallas import tpu_sc as plsc

N = 10000
D = 128
E = 320000
A = 16
L = 3
N_AGENTS = 4

NC = 2
NS = 16
EW = E // (NC * NS)
C = 128
EW_PAD = 10240
NCH = EW_PAD // C
CC = 80
NCHC = EW // CC
N_PAD = 10240
RPT = N_PAD // NS
ZCH = 16
PKM = 16384

_MESH = plsc.VectorSubcoreMesh(core_axis_name="c", subcore_axis_name="s")

_CP = pltpu.CompilerParams()
if "needs_layout_passes" in pltpu.CompilerParams.__dataclass_fields__:
    _CP = dataclasses.replace(_CP, needs_layout_passes=False)


def _sc_count(dst4):
    @functools.partial(
        pl.kernel,
        out_type=jax.ShapeDtypeStruct((NC, NS, N), jnp.float32),
        mesh=_MESH,
        compiler_params=_CP,
        scratch_types=[
            pltpu.VMEM((NCHC, CC), jnp.int32),
            pltpu.VMEM((N,), jnp.float32),
        ],
    )
    def k(dst_hbm, out_hbm, dst_v, cnt_v):
        c = lax.axis_index("c")
        s = lax.axis_index("s")
        pltpu.sync_copy(dst_hbm.at[c, s], dst_v)
        z16 = jnp.zeros((16,), jnp.float32)

        @pl.loop(0, N, step=16)
        def _(i):
            cnt_v[pl.ds(i, 16)] = z16

        ones = jnp.ones((16,), jnp.float32)

        @pl.loop(0, NCHC)
        def _(j):
            @pl.loop(0, CC, step=16)
            def _(t):
                idx = dst_v[j, pl.ds(t, 16)]
                plsc.addupdate_scatter(cnt_v, [idx], ones)

        pltpu.sync_copy(cnt_v, out_hbm.at[c, s])

    return k(dst4)


def _sc_segsum(h, pk4):
    @functools.partial(
        pl.kernel,
        out_type=jax.ShapeDtypeStruct((NC, N_PAD, D), jnp.float32),
        mesh=_MESH,
        scratch_types=[
            pltpu.VMEM((NCH, C), jnp.int32),
            pltpu.VMEM((2, C), jnp.int32),
            pltpu.VMEM((2, C), jnp.int32),
            pltpu.VMEM((C, D), jnp.float32),
            pltpu.VMEM((C, D), jnp.float32),
            pltpu.VMEM((ZCH, D), jnp.float32),
            pltpu.VMEM_SHARED((N_PAD, D), jnp.float32),
            pltpu.SemaphoreType.DMA,
            pltpu.SemaphoreType.DMA,
            pltpu.SemaphoreType.DMA,
            pltpu.SemaphoreType.DMA,
            pltpu.SemaphoreType.DMA,
        ],
    )
    def k(h_hbm, pk_hbm, out_hbm, pk_v, sib, dib, g0, g1, zbuf, acc,
          zsem, gs0, gs1, ss0, ss1):
        c = lax.axis_index("c")
        s = lax.axis_index("s")
        z16 = jnp.zeros((16,), jnp.float32)

        @pl.loop(0, ZCH)
        def _(r):
            @pl.loop(0, D, step=16)
            def _(t):
                zbuf[r, pl.ds(t, 16)] = z16

        row0 = s * RPT

        @pl.loop(0, RPT // ZCH)
        def _(kk):
            pltpu.async_copy(zbuf, acc.at[pl.ds(row0 + ZCH * kk, ZCH)], zsem)

        @pl.loop(0, RPT // ZCH)
        def _(kk):
            pltpu.make_async_copy(zbuf, acc.at[pl.ds(row0, ZCH)], zsem).wait()

        pltpu.sync_copy(pk_hbm.at[c, s], pk_v)
        plsc.subcore_barrier()

        bufs = ((g0, gs0, ss0), (g1, gs1, ss1))

        def unpack(j, bb):
            @pl.loop(0, C, step=16)
            def _(t):
                v = pk_v[j, pl.ds(t, 16)]
                sib[bb, pl.ds(t, 16)] = lax.bitwise_and(v, PKM - 1)
                dib[bb, pl.ds(t, 16)] = lax.shift_right_logical(v, 14)

        def start_gather(bb, g, gs):
            pltpu.async_copy(h_hbm.at[sib.at[bb]], g, gs)

        for bb in range(2):
            unpack(bb, bb)
            start_gather(bb, bufs[bb][0], bufs[bb][1])

        @pl.loop(0, NCH, step=2)
        def _(j):
            for bb in range(2):
                g, gs, ss = bufs[bb]
                jb = j + bb
                pltpu.make_async_copy(h_hbm.at[sib.at[bb]], g, gs).wait()
                pltpu.async_copy(g, acc.at[dib.at[bb]], ss, add=True)
                pltpu.make_async_copy(g, acc.at[dib.at[bb]], ss).wait()

                @pl.when(jb + 2 < NCH)
                def _():
                    unpack(jb + 2, bb)
                    start_gather(bb, g, gs)

        plsc.subcore_barrier()
        sl = pl.ds(row0, RPT)
        pltpu.sync_copy(acc.at[sl], out_hbm.at[c, sl])

    return k(h, pk4)


_R = 2000


def _tc_layer(h, m, cnt_t, ws, wn, bias):
    def body(h_ref, m0_ref, m1_ref, cnt_ref, ws_ref, wn_ref, b_ref, o_ref):
        cnt = jnp.sum(cnt_ref[...], axis=1)
        inv = 1.0 / jnp.maximum(cnt, 1.0)
        mm = (m0_ref[0] + m1_ref[0]) * inv[:, None]
        acc = jnp.dot(h_ref[...], ws_ref[...], preferred_element_type=jnp.float32)
        acc = acc + jnp.dot(mm, wn_ref[...], preferred_element_type=jnp.float32)
        o_ref[...] = jnp.maximum(acc + b_ref[...], 0.0)

    return pl.pallas_call(
        body,
        grid=(N // _R,),
        in_specs=[
            pl.BlockSpec((_R, D), lambda i: (i, 0)),
            pl.BlockSpec((1, _R, D), lambda i: (0, i, 0)),
            pl.BlockSpec((1, _R, D), lambda i: (1, i, 0)),
            pl.BlockSpec((_R, NC * NS), lambda i: (i, 0)),
            pl.BlockSpec((D, D), lambda i: (0, 0)),
            pl.BlockSpec((D, D), lambda i: (0, 0)),
            pl.BlockSpec((1, D), lambda i: (0, 0)),
        ],
        out_specs=pl.BlockSpec((_R, D), lambda i: (i, 0)),
        out_shape=jax.ShapeDtypeStruct((N, D), jnp.float32),
    )(h, m, m, cnt_t, ws, wn, bias.reshape(1, D))


def _tc_heads(h, w_heads, b_heads):
    def body(h_ref, w_ref, b_ref, o_ref):
        o_ref[...] = (
            jnp.dot(h_ref[...], w_ref[...], preferred_element_type=jnp.float32)
            + b_ref[...]
        )

    return pl.pallas_call(
        body,
        grid=(N // _R,),
        in_specs=[
            pl.BlockSpec((_R, D), lambda i: (i, 0)),
            pl.BlockSpec((D, A + 1), lambda i: (0, 0)),
            pl.BlockSpec((1, A + 1), lambda i: (0, 0)),
        ],
        out_specs=pl.BlockSpec((_R, A + 1), lambda i: (i, 0)),
        out_shape=jax.ShapeDtypeStruct((N, A + 1), jnp.float32),
    )(h, w_heads, b_heads.reshape(1, A + 1))


def kernel(x0, x1, x2, x3, edge_index0, edge_index1, edge_index2, edge_index3,
           Wself, Wnbr, b, Wp, bp, Wv, bv):
    xs = [x0, x1, x2, x3]
    eis = [edge_index0, edge_index1, edge_index2, edge_index3]

    pks, cnts = [], []
    pad_s = jnp.zeros((NC * NS, EW_PAD - EW), jnp.int32)
    pad_d = jnp.full((NC * NS, EW_PAD - EW), N_PAD - 1, jnp.int32)
    for i in range(N_AGENTS):
        src_p = jnp.concatenate([eis[i][0].reshape(NC * NS, EW), pad_s], axis=1)
        dst_p = jnp.concatenate([eis[i][1].reshape(NC * NS, EW), pad_d], axis=1)
        pks.append((dst_p * PKM + src_p).reshape(NC, NS, NCH, C))
    for i in range(N_AGENTS):
        cp = _sc_count(eis[i][1].reshape(NC, NS, NCHC, CC))
        cnts.append(cp.reshape(NC * NS, N).T)

    hs = list(xs)
    for l in range(L):
        ms = [_sc_segsum(hs[i], pks[i]) for i in range(N_AGENTS)]
        hs = [
            _tc_layer(hs[i], ms[i], cnts[i], Wself[i, l], Wnbr[i, l], b[i, l])
            for i in range(N_AGENTS)
        ]

    logits, values = [], []
    for i in range(N_AGENTS):
        wh = jnp.concatenate([Wp[i], Wv[i]], axis=1)
        bh = jnp.concatenate([bp[i], bv[i]], axis=0)
        out = _tc_heads(hs[i], wh, bh)
        logits.append(out[:, :A])
        values.append(out[:, A:])
    return (jnp.stack(logits, axis=0), jnp.stack(values, axis=0))


# speedup vs baseline: 1.0005x; 1.0005x over previous
import dataclasses
import functools

import jax
import jax.numpy as jnp
from jax import lax
from jax.experimental import pallas as pl
from jax.experimental.pallas import tpu as pltpu
from jax.experimental.pallas import tpu_sc as plsc

N = 10000
D = 128
E = 320000
A = 16
L = 3
N_AGENTS = 4

NC = 2
NS = 16
EW = E // (NC * NS)
C = 128
EW_PAD = 10240
NCH = EW_PAD // C
CC = 80
NCHC = EW // CC
N_PAD = 10240
RPT = N_PAD // NS
ZCH = 16
PKM = 16384

_MESH = plsc.VectorSubcoreMesh(core_axis_name="c", subcore_axis_name="s")

_CP = pltpu.CompilerParams()
if "needs_layout_passes" in pltpu.CompilerParams.__dataclass_fields__:
    _CP = dataclasses.replace(_CP, needs_layout_passes=False)


def _sc_count(dst4):
    @functools.partial(
        pl.kernel,
        out_type=jax.ShapeDtypeStruct((NC, NS, N), jnp.float32),
        mesh=_MESH,
        compiler_params=_CP,
        scratch_types=[
            pltpu.VMEM((NCHC, CC), jnp.int32),
            pltpu.VMEM((N,), jnp.float32),
        ],
    )
    def k(dst_hbm, out_hbm, dst_v, cnt_v):
        c = lax.axis_index("c")
        s = lax.axis_index("s")
        pltpu.sync_copy(dst_hbm.at[c, s], dst_v)
        z16 = jnp.zeros((16,), jnp.float32)

        @pl.loop(0, N, step=16)
        def _(i):
            cnt_v[pl.ds(i, 16)] = z16

        ones = jnp.ones((16,), jnp.float32)

        @pl.loop(0, NCHC)
        def _(j):
            @pl.loop(0, CC, step=16)
            def _(t):
                idx = dst_v[j, pl.ds(t, 16)]
                plsc.addupdate_scatter(cnt_v, [idx], ones)

        pltpu.sync_copy(cnt_v, out_hbm.at[c, s])

    return k(dst4)


def _sc_segsum(h, pk4):
    @functools.partial(
        pl.kernel,
        out_type=jax.ShapeDtypeStruct((NC, N_PAD, D), jnp.float32),
        mesh=_MESH,
        scratch_types=[
            pltpu.VMEM((NCH, C), jnp.int32),
            pltpu.VMEM((2, C), jnp.int32),
            pltpu.VMEM((2, C), jnp.int32),
            pltpu.VMEM((C, D), jnp.float32),
            pltpu.VMEM((C, D), jnp.float32),
            pltpu.VMEM((ZCH, D), jnp.float32),
            pltpu.VMEM_SHARED((N_PAD, D), jnp.float32),
            pltpu.SemaphoreType.DMA,
            pltpu.SemaphoreType.DMA,
            pltpu.SemaphoreType.DMA,
            pltpu.SemaphoreType.DMA,
            pltpu.SemaphoreType.DMA,
        ],
    )
    def k(h_hbm, pk_hbm, out_hbm, pk_v, sib, dib, g0, g1, zbuf, acc,
          zsem, gs0, gs1, ss0, ss1):
        c = lax.axis_index("c")
        s = lax.axis_index("s")
        z16 = jnp.zeros((16,), jnp.float32)

        @pl.loop(0, ZCH)
        def _(r):
            @pl.loop(0, D, step=16)
            def _(t):
                zbuf[r, pl.ds(t, 16)] = z16

        row0 = s * RPT

        @pl.loop(0, RPT // ZCH)
        def _(kk):
            pltpu.async_copy(zbuf, acc.at[pl.ds(row0 + ZCH * kk, ZCH)], zsem)

        @pl.loop(0, RPT // ZCH)
        def _(kk):
            pltpu.make_async_copy(zbuf, acc.at[pl.ds(row0, ZCH)], zsem).wait()

        pltpu.sync_copy(pk_hbm.at[c, s], pk_v)
        plsc.subcore_barrier()

        bufs = ((g0, gs0, ss0), (g1, gs1, ss1))

        def unpack(j, bb):
            @pl.loop(0, C, step=16)
            def _(t):
                v = pk_v[j, pl.ds(t, 16)]
                sib[bb, pl.ds(t, 16)] = lax.bitwise_and(v, PKM - 1)
                dib[bb, pl.ds(t, 16)] = lax.shift_right_logical(v, 14)

        def start_gather(bb, g, gs):
            pltpu.async_copy(h_hbm.at[sib.at[bb]], g, gs)

        for bb in range(2):
            unpack(bb, bb)
            start_gather(bb, bufs[bb][0], bufs[bb][1])

        @pl.loop(0, NCH, step=2)
        def _(j):
            for bb in range(2):
                g, gs, ss = bufs[bb]
                jb = j + bb
                pltpu.make_async_copy(h_hbm.at[pl.ds(0, C)], g, gs).wait()
                pltpu.async_copy(g, acc.at[dib.at[bb]], ss, add=True)
                pltpu.make_async_copy(h_hbm.at[pl.ds(0, C)], g, ss).wait()

                @pl.when(jb + 2 < NCH)
                def _():
                    unpack(jb + 2, bb)
                    start_gather(bb, g, gs)

        plsc.subcore_barrier()
        sl = pl.ds(row0, RPT)
        pltpu.sync_copy(acc.at[sl], out_hbm.at[c, sl])

    return k(h, pk4)


_R = 2000


def _tc_layer(h, m, cnt_t, ws, wn, bias):
    def body(h_ref, m0_ref, m1_ref, cnt_ref, ws_ref, wn_ref, b_ref, o_ref):
        cnt = jnp.sum(cnt_ref[...], axis=1)
        inv = 1.0 / jnp.maximum(cnt, 1.0)
        mm = (m0_ref[0] + m1_ref[0]) * inv[:, None]
        acc = jnp.dot(h_ref[...], ws_ref[...], preferred_element_type=jnp.float32)
        acc = acc + jnp.dot(mm, wn_ref[...], preferred_element_type=jnp.float32)
        o_ref[...] = jnp.maximum(acc + b_ref[...], 0.0)

    return pl.pallas_call(
        body,
        grid=(N // _R,),
        in_specs=[
            pl.BlockSpec((_R, D), lambda i: (i, 0)),
            pl.BlockSpec((1, _R, D), lambda i: (0, i, 0)),
            pl.BlockSpec((1, _R, D), lambda i: (1, i, 0)),
            pl.BlockSpec((_R, NC * NS), lambda i: (i, 0)),
            pl.BlockSpec((D, D), lambda i: (0, 0)),
            pl.BlockSpec((D, D), lambda i: (0, 0)),
            pl.BlockSpec((1, D), lambda i: (0, 0)),
        ],
        out_specs=pl.BlockSpec((_R, D), lambda i: (i, 0)),
        out_shape=jax.ShapeDtypeStruct((N, D), jnp.float32),
    )(h, m, m, cnt_t, ws, wn, bias.reshape(1, D))


def _tc_heads(h, w_heads, b_heads):
    def body(h_ref, w_ref, b_ref, o_ref):
        o_ref[...] = (
            jnp.dot(h_ref[...], w_ref[...], preferred_element_type=jnp.float32)
            + b_ref[...]
        )

    return pl.pallas_call(
        body,
        grid=(N // _R,),
        in_specs=[
            pl.BlockSpec((_R, D), lambda i: (i, 0)),
            pl.BlockSpec((D, A + 1), lambda i: (0, 0)),
            pl.BlockSpec((1, A + 1), lambda i: (0, 0)),
        ],
        out_specs=pl.BlockSpec((_R, A + 1), lambda i: (i, 0)),
        out_shape=jax.ShapeDtypeStruct((N, A + 1), jnp.float32),
    )(h, w_heads, b_heads.reshape(1, A + 1))


def kernel(x0, x1, x2, x3, edge_index0, edge_index1, edge_index2, edge_index3,
           Wself, Wnbr, b, Wp, bp, Wv, bv):
    xs = [x0, x1, x2, x3]
    eis = [edge_index0, edge_index1, edge_index2, edge_index3]

    pks, cnts = [], []
    pad_s = jnp.zeros((NC * NS, EW_PAD - EW), jnp.int32)
    pad_d = jnp.full((NC * NS, EW_PAD - EW), N_PAD - 1, jnp.int32)
    for i in range(N_AGENTS):
        src_p = jnp.concatenate([eis[i][0].reshape(NC * NS, EW), pad_s], axis=1)
        dst_p = jnp.concatenate([eis[i][1].reshape(NC * NS, EW), pad_d], axis=1)
        pks.append((dst_p * PKM + src_p).reshape(NC, NS, NCH, C))
    for i in range(N_AGENTS):
        cp = _sc_count(eis[i][1].reshape(NC, NS, NCHC, CC))
        cnts.append(cp.reshape(NC * NS, N).T)

    hs = list(xs)
    for l in range(L):
        ms = [_sc_segsum(hs[i], pks[i]) for i in range(N_AGENTS)]
        hs = [
            _tc_layer(hs[i], ms[i], cnts[i], Wself[i, l], Wnbr[i, l], b[i, l])
            for i in range(N_AGENTS)
        ]

    logits, values = [], []
    for i in range(N_AGENTS):
        wh = jnp.concatenate([Wp[i], Wv[i]], axis=1)
        bh = jnp.concatenate([bp[i], bv[i]], axis=0)
        out = _tc_heads(hs[i], wh, bh)
        logits.append(out[:, :A])
        values.append(out[:, A:])
    return (jnp.stack(logits, axis=0), jnp.stack(values, axis=0))
